# baseline (speedup 1.0000x reference)
import jax
import jax.numpy as jnp
from jax import lax
from jax.experimental import pallas as pl
from jax.experimental.pallas import tpu as pltpu
from jax.experimental.pallas import tpu_sc as plsc

B = 16384
D_NUM = 13
N_CAT = 26
CARD = 1000
D_TOKEN = 64
TOTAL_CAT = N_CAT * CARD
N_TOK = 1 + D_NUM + N_CAT

NUM_ROWS = (1 + D_NUM) * D_TOKEN
OUT_ROWS = N_TOK * D_TOKEN

SC_CORES = 2
SC_SUBCORES = 16
NW = SC_CORES * SC_SUBCORES
COLS_PER_W = B // NW
BLK = 128
N_BLK = COLS_PER_W // BLK
W_IDX = N_CAT * COLS_PER_W


TAB_W = 128


def _prep_kernel(emb_ref, bias_ref, xcat_ref, tab_ref, idx_ref):
    tab_ref[...] = emb_ref[...] + bias_ref[...]
    offs = lax.broadcasted_iota(jnp.int32, (N_CAT, B), 0) * CARD
    idx_ref[...] = xcat_ref[...] + offs


def _prep(cat_emb_p, bias_cat_p, x_cat_t):
    emb3 = cat_emb_p.reshape(N_CAT, CARD, TAB_W)
    bias3 = bias_cat_p.reshape(N_CAT, 1, TAB_W)
    tab3, idx = pl.pallas_call(
        _prep_kernel,
        out_shape=(
            jax.ShapeDtypeStruct((N_CAT, CARD, TAB_W), jnp.float32),
            jax.ShapeDtypeStruct((N_CAT, B), jnp.int32),
        ),
    )(emb3, bias3, x_cat_t)
    return tab3.reshape(TOTAL_CAT, TAB_W), idx.reshape(N_CAT * B)


def _sc_body(tab_ref, idx_ref, out_ref, idx_v, rows_v, tp_v, sem):
    wid = lax.axis_index("s") * SC_CORES + lax.axis_index("c")
    col0 = wid * COLS_PER_W

    def stage(c, carry):
        pltpu.sync_copy(
            idx_ref.at[pl.ds(c * B + col0, COLS_PER_W)],
            idx_v.at[pl.ds(c * COLS_PER_W, COLS_PER_W)],
        )
        return carry

    lax.fori_loop(0, N_CAT, stage, 0)

    iotas = [lax.iota(jnp.int32, 16) + 16 * ic for ic in range(BLK // 16)]

    def unit(u, carry):
        c = u // N_BLK
        j = u % N_BLK
        pltpu.async_copy(
            tab_ref.at[idx_v.at[pl.ds(c * COLS_PER_W + j * BLK, BLK)]],
            rows_v,
            sem,
        ).wait()
        for d in range(D_TOKEN):
            dvec = jnp.full((16,), d, jnp.int32)
            for ic in range(BLK // 16):
                tp_v[d, pl.ds(16 * ic, 16)] = plsc.load_gather(
                    rows_v, [iotas[ic], dvec]
                )
        pltpu.sync_copy(
            tp_v,
            out_ref.at[
                pl.ds(NUM_ROWS + c * D_TOKEN, D_TOKEN),
                pl.ds(col0 + j * BLK, BLK),
            ],
        )
        return carry

    lax.fori_loop(0, N_CAT * N_BLK, unit, 0)


def _sc_gather(table, idx_flat):
    mesh = plsc.VectorSubcoreMesh(core_axis_name="c", subcore_axis_name="s")
    f = pl.kernel(
        _sc_body,
        out_type=jax.ShapeDtypeStruct((OUT_ROWS, B), jnp.float32),
        mesh=mesh,
        scratch_types=[
            pltpu.VMEM((W_IDX,), jnp.int32),
            pltpu.VMEM((BLK, TAB_W), jnp.float32),
            pltpu.VMEM((D_TOKEN, BLK), jnp.float32),
            pltpu.SemaphoreType.DMA,
        ],
        compiler_params=pltpu.CompilerParams(needs_layout_passes=False),
    )
    return f(table, idx_flat)


def _num_kernel(x_ref, w_ref, b_ref, alias_ref, o_ref):
    del alias_ref
    o_ref[...] = (
        w_ref[0, 0][:, None] * x_ref[0, 0][None, :] + b_ref[0, 0][:, None]
    )


def _num_finish(xf_t, weight, bias_num, out0):
    bs = 1024
    w3 = weight.reshape(1 + D_NUM, 1, D_TOKEN)
    b3 = bias_num.reshape(1 + D_NUM, 1, D_TOKEN)
    xf3 = xf_t.reshape(1 + D_NUM, 1, B)
    return pl.pallas_call(
        _num_kernel,
        grid=(1 + D_NUM, B // bs),
        in_specs=[
            pl.BlockSpec((1, 1, bs), lambda t, i: (t, 0, i)),
            pl.BlockSpec((1, 1, D_TOKEN), lambda t, i: (t, 0, 0)),
            pl.BlockSpec((1, 1, D_TOKEN), lambda t, i: (t, 0, 0)),
            pl.BlockSpec(memory_space=pl.ANY),
        ],
        out_specs=pl.BlockSpec((D_TOKEN, bs), lambda t, i: (t, i)),
        out_shape=jax.ShapeDtypeStruct((OUT_ROWS, B), jnp.float32),
        input_output_aliases={3: 0},
    )(xf3, w3, b3, out0)


def kernel(x_num, x_cat, weight, bias, cat_emb):
    x_cat_t = x_cat.T
    xf_t = jnp.concatenate(
        [jnp.ones((1, B), jnp.float32), x_num.T], axis=0
    )
    bias_num = jnp.concatenate(
        [jnp.zeros((1, D_TOKEN), jnp.float32), bias[:D_NUM]], axis=0
    )
    cat_emb_p = jnp.pad(cat_emb, ((0, 0), (0, TAB_W - D_TOKEN)))
    bias_cat_p = jnp.pad(bias[D_NUM:], ((0, 0), (0, TAB_W - D_TOKEN)))
    table, idx_flat = _prep(cat_emb_p, bias_cat_p, x_cat_t)
    out0 = _sc_gather(table, idx_flat)
    out2d = _num_finish(xf_t, weight, bias_num, out0)
    return out2d.reshape(N_TOK, D_TOKEN, B).transpose(2, 0, 1)


# speedup vs baseline: 1.1095x; 1.1095x over previous
import jax
import jax.numpy as jnp
from jax import lax
from jax.experimental import pallas as pl
from jax.experimental.pallas import tpu as pltpu
from jax.experimental.pallas import tpu_sc as plsc

B = 16384
D_NUM = 13
N_CAT = 26
CARD = 1000
D_TOKEN = 64
TOTAL_CAT = N_CAT * CARD
N_TOK = 1 + D_NUM + N_CAT

NUM_ROWS = (1 + D_NUM) * D_TOKEN
OUT_ROWS = N_TOK * D_TOKEN

SC_CORES = 2
SC_SUBCORES = 16
NW = SC_CORES * SC_SUBCORES
COLS_PER_W = B // NW
BLK = 128
N_BLK = COLS_PER_W // BLK
W_IDX = N_CAT * COLS_PER_W


TAB_W = 128


def _prep_kernel(emb_ref, bias_ref, xcat_ref, tab_ref, idx_ref):
    tab_ref[...] = emb_ref[...] + bias_ref[...]
    offs = lax.broadcasted_iota(jnp.int32, (N_CAT, B), 0) * CARD
    idx_ref[...] = xcat_ref[...] + offs


def _prep(cat_emb_p, bias_cat_p, x_cat_t):
    emb3 = cat_emb_p.reshape(N_CAT, CARD, TAB_W)
    bias3 = bias_cat_p.reshape(N_CAT, 1, TAB_W)
    tab3, idx = pl.pallas_call(
        _prep_kernel,
        out_shape=(
            jax.ShapeDtypeStruct((N_CAT, CARD, TAB_W), jnp.float32),
            jax.ShapeDtypeStruct((N_CAT, B), jnp.int32),
        ),
    )(emb3, bias3, x_cat_t)
    return tab3.reshape(TOTAL_CAT, TAB_W), idx.reshape(N_CAT * B)


def _sc_body(tab_ref, idx_ref, out_ref, idx_v, rows_v, tp_v, sem):
    wid = lax.axis_index("s") * SC_CORES + lax.axis_index("c")
    col0 = wid * COLS_PER_W

    def stage(c, carry):
        pltpu.sync_copy(
            idx_ref.at[pl.ds(c * B + col0, COLS_PER_W)],
            idx_v.at[pl.ds(c * COLS_PER_W, COLS_PER_W)],
        )
        return carry

    lax.fori_loop(0, N_CAT, stage, 0)

    d_iotas = [lax.iota(jnp.int32, 16) + 16 * dk for dk in range(D_TOKEN // 16)]

    def unit(u, carry):
        c = u // N_BLK
        j = u % N_BLK
        pltpu.async_copy(
            tab_ref.at[idx_v.at[pl.ds(c * COLS_PER_W + j * BLK, BLK)]],
            rows_v,
            sem,
        ).wait()
        for r in range(BLK):
            rvec = jnp.full((16,), r, jnp.int32)
            for dk in range(D_TOKEN // 16):
                plsc.store_scatter(
                    tp_v,
                    [d_iotas[dk], rvec],
                    rows_v[r, pl.ds(16 * dk, 16)],
                )
        pltpu.sync_copy(
            tp_v.at[:, pl.ds(0, BLK)],
            out_ref.at[
                pl.ds(NUM_ROWS + c * D_TOKEN, D_TOKEN),
                pl.ds(col0 + j * BLK, BLK),
            ],
        )
        return carry

    lax.fori_loop(0, N_CAT * N_BLK, unit, 0)


def _sc_gather(table, idx_flat):
    mesh = plsc.VectorSubcoreMesh(core_axis_name="c", subcore_axis_name="s")
    f = pl.kernel(
        _sc_body,
        out_type=jax.ShapeDtypeStruct((OUT_ROWS, B), jnp.float32),
        mesh=mesh,
        scratch_types=[
            pltpu.VMEM((W_IDX,), jnp.int32),
            pltpu.VMEM((BLK, TAB_W), jnp.float32),
            pltpu.VMEM((D_TOKEN, BLK + 1), jnp.float32),
            pltpu.SemaphoreType.DMA,
        ],
        compiler_params=pltpu.CompilerParams(needs_layout_passes=False),
    )
    return f(table, idx_flat)


def _num_kernel(x_ref, w_ref, b_ref, alias_ref, o_ref):
    del alias_ref
    o_ref[...] = (
        w_ref[0, 0][:, None] * x_ref[0, 0][None, :] + b_ref[0, 0][:, None]
    )


def _num_finish(xf_t, weight, bias_num, out0):
    bs = 1024
    w3 = weight.reshape(1 + D_NUM, 1, D_TOKEN)
    b3 = bias_num.reshape(1 + D_NUM, 1, D_TOKEN)
    xf3 = xf_t.reshape(1 + D_NUM, 1, B)
    return pl.pallas_call(
        _num_kernel,
        grid=(1 + D_NUM, B // bs),
        in_specs=[
            pl.BlockSpec((1, 1, bs), lambda t, i: (t, 0, i)),
            pl.BlockSpec((1, 1, D_TOKEN), lambda t, i: (t, 0, 0)),
            pl.BlockSpec((1, 1, D_TOKEN), lambda t, i: (t, 0, 0)),
            pl.BlockSpec(memory_space=pl.ANY),
        ],
        out_specs=pl.BlockSpec((D_TOKEN, bs), lambda t, i: (t, i)),
        out_shape=jax.ShapeDtypeStruct((OUT_ROWS, B), jnp.float32),
        input_output_aliases={3: 0},
    )(xf3, w3, b3, out0)


def kernel(x_num, x_cat, weight, bias, cat_emb):
    x_cat_t = x_cat.T
    xf_t = jnp.concatenate(
        [jnp.ones((1, B), jnp.float32), x_num.T], axis=0
    )
    bias_num = jnp.concatenate(
        [jnp.zeros((1, D_TOKEN), jnp.float32), bias[:D_NUM]], axis=0
    )
    cat_emb_p = jnp.pad(cat_emb, ((0, 0), (0, TAB_W - D_TOKEN)))
    bias_cat_p = jnp.pad(bias[D_NUM:], ((0, 0), (0, TAB_W - D_TOKEN)))
    table, idx_flat = _prep(cat_emb_p, bias_cat_p, x_cat_t)
    out0 = _sc_gather(table, idx_flat)
    out2d = _num_finish(xf_t, weight, bias_num, out0)
    return out2d.reshape(N_TOK, D_TOKEN, B).transpose(2, 0, 1)


# speedup vs baseline: 1.4443x; 1.3018x over previous
import jax
import jax.numpy as jnp
from jax import lax
from jax.experimental import pallas as pl
from jax.experimental.pallas import tpu as pltpu
from jax.experimental.pallas import tpu_sc as plsc

B = 16384
D_NUM = 13
N_CAT = 26
CARD = 1000
D_TOKEN = 64
TOTAL_CAT = N_CAT * CARD
N_TOK = 1 + D_NUM + N_CAT

NUM_ROWS = (1 + D_NUM) * D_TOKEN
OUT_ROWS = N_TOK * D_TOKEN

SC_CORES = 2
SC_SUBCORES = 16
NW = SC_CORES * SC_SUBCORES
COLS_PER_W = B // NW
BLK = 256
N_BLK = COLS_PER_W // BLK
N_UNIT = N_CAT * N_BLK
W_IDX = N_CAT * COLS_PER_W


TAB_W = 128


def _prep_kernel(emb_ref, bias_ref, xcat_ref, tab_ref, idx_ref):
    tab_ref[...] = emb_ref[...] + bias_ref[...]
    offs = lax.broadcasted_iota(jnp.int32, (N_CAT, B), 0) * CARD
    idx_ref[...] = xcat_ref[...] + offs


def _prep(cat_emb_p, bias_cat_p, x_cat_t):
    emb3 = cat_emb_p.reshape(N_CAT, CARD, TAB_W)
    bias3 = bias_cat_p.reshape(N_CAT, 1, TAB_W)
    tab3, idx = pl.pallas_call(
        _prep_kernel,
        out_shape=(
            jax.ShapeDtypeStruct((N_CAT, CARD, TAB_W), jnp.float32),
            jax.ShapeDtypeStruct((N_CAT, B), jnp.int32),
        ),
    )(emb3, bias3, x_cat_t)
    return tab3.reshape(TOTAL_CAT, TAB_W), idx.reshape(N_CAT * B)


def _sc_body(
    tab_ref,
    idx_ref,
    out_ref,
    idx_v,
    rows_a,
    rows_b,
    tp_a,
    tp_b,
    sg_a,
    sg_b,
    so_a,
    so_b,
):
    wid = lax.axis_index("s") * SC_CORES + lax.axis_index("c")
    col0 = wid * COLS_PER_W

    def stage(c, carry):
        pltpu.sync_copy(
            idx_ref.at[pl.ds(c * B + col0, COLS_PER_W)],
            idx_v.at[pl.ds(c * COLS_PER_W, COLS_PER_W)],
        )
        return carry

    lax.fori_loop(0, N_CAT, stage, 0)

    d_iotas = [lax.iota(jnp.int32, 16) + 16 * dk for dk in range(D_TOKEN // 16)]

    def start_gather(u, rows_v, sem):
        c = u // N_BLK
        j = u % N_BLK
        pltpu.async_copy(
            tab_ref.at[idx_v.at[pl.ds(c * COLS_PER_W + j * BLK, BLK)]],
            rows_v,
            sem,
        )

    def transpose(rows_v, tp_v):
        def rchunk(rc, carry):
            for k in range(8):
                r = rc * 8 + k
                rvec = jnp.full((16,), r, jnp.int32)
                for dk in range(D_TOKEN // 16):
                    plsc.store_scatter(
                        tp_v,
                        [d_iotas[dk], rvec],
                        rows_v[r, pl.ds(16 * dk, 16)],
                    )
            return carry

        lax.fori_loop(0, BLK // 8, rchunk, 0)

    def out_slice(u):
        c = u // N_BLK
        j = u % N_BLK
        return out_ref.at[
            pl.ds(NUM_ROWS + c * D_TOKEN, D_TOKEN),
            pl.ds(col0 + j * BLK, BLK),
        ]

    def start_out(u, tp_v, sem):
        pltpu.async_copy(tp_v.at[:, pl.ds(0, BLK)], out_slice(u), sem)

    def wait_out(u, tp_v, sem):
        pltpu.make_async_copy(tp_v.at[:, pl.ds(0, BLK)], out_slice(u), sem).wait()

    def wait_gather(rows_v, sem):
        pltpu.make_async_copy(
            tab_ref.at[idx_v.at[pl.ds(0, BLK)]], rows_v, sem
        ).wait()

    start_gather(0, rows_a, sg_a)

    def pair(p, carry):
        u0 = 2 * p
        u1 = u0 + 1
        start_gather(u1, rows_b, sg_b)
        wait_gather(rows_a, sg_a)

        @pl.when(p > 0)
        def _():
            wait_out(u0, tp_a, so_a)

        transpose(rows_a, tp_a)
        start_out(u0, tp_a, so_a)

        @pl.when(p < N_UNIT // 2 - 1)
        def _():
            start_gather(u0 + 2, rows_a, sg_a)

        wait_gather(rows_b, sg_b)

        @pl.when(p > 0)
        def _():
            wait_out(u1, tp_b, so_b)

        transpose(rows_b, tp_b)
        start_out(u1, tp_b, so_b)
        return carry

    lax.fori_loop(0, N_UNIT // 2, pair, 0)
    wait_out(0, tp_a, so_a)
    wait_out(0, tp_b, so_b)


def _sc_gather(table, idx_flat):
    mesh = plsc.VectorSubcoreMesh(core_axis_name="c", subcore_axis_name="s")
    f = pl.kernel(
        _sc_body,
        out_type=jax.ShapeDtypeStruct((OUT_ROWS, B), jnp.float32),
        mesh=mesh,
        scratch_types=[
            pltpu.VMEM((W_IDX,), jnp.int32),
            pltpu.VMEM((BLK, TAB_W), jnp.float32),
            pltpu.VMEM((BLK, TAB_W), jnp.float32),
            pltpu.VMEM((D_TOKEN, BLK + 1), jnp.float32),
            pltpu.VMEM((D_TOKEN, BLK + 1), jnp.float32),
            pltpu.SemaphoreType.DMA,
            pltpu.SemaphoreType.DMA,
            pltpu.SemaphoreType.DMA,
            pltpu.SemaphoreType.DMA,
        ],
        compiler_params=pltpu.CompilerParams(needs_layout_passes=False),
    )
    return f(table, idx_flat)


def _num_kernel(x_ref, w_ref, b_ref, alias_ref, o_ref):
    del alias_ref
    o_ref[...] = (
        w_ref[0, 0][:, None] * x_ref[0, 0][None, :] + b_ref[0, 0][:, None]
    )


def _num_finish(xf_t, weight, bias_num, out0):
    bs = 1024
    w3 = weight.reshape(1 + D_NUM, 1, D_TOKEN)
    b3 = bias_num.reshape(1 + D_NUM, 1, D_TOKEN)
    xf3 = xf_t.reshape(1 + D_NUM, 1, B)
    return pl.pallas_call(
        _num_kernel,
        grid=(1 + D_NUM, B // bs),
        in_specs=[
            pl.BlockSpec((1, 1, bs), lambda t, i: (t, 0, i)),
            pl.BlockSpec((1, 1, D_TOKEN), lambda t, i: (t, 0, 0)),
            pl.BlockSpec((1, 1, D_TOKEN), lambda t, i: (t, 0, 0)),
            pl.BlockSpec(memory_space=pl.ANY),
        ],
        out_specs=pl.BlockSpec((D_TOKEN, bs), lambda t, i: (t, i)),
        out_shape=jax.ShapeDtypeStruct((OUT_ROWS, B), jnp.float32),
        input_output_aliases={3: 0},
    )(xf3, w3, b3, out0)


def kernel(x_num, x_cat, weight, bias, cat_emb):
    x_cat_t = x_cat.T
    xf_t = jnp.concatenate(
        [jnp.ones((1, B), jnp.float32), x_num.T], axis=0
    )
    bias_num = jnp.concatenate(
        [jnp.zeros((1, D_TOKEN), jnp.float32), bias[:D_NUM]], axis=0
    )
    cat_emb_p = jnp.pad(cat_emb, ((0, 0), (0, TAB_W - D_TOKEN)))
    bias_cat_p = jnp.pad(bias[D_NUM:], ((0, 0), (0, TAB_W - D_TOKEN)))
    table, idx_flat = _prep(cat_emb_p, bias_cat_p, x_cat_t)
    out0 = _sc_gather(table, idx_flat)
    out2d = _num_finish(xf_t, weight, bias_num, out0)
    return out2d.reshape(N_TOK, D_TOKEN, B).transpose(2, 0, 1)


# speedup vs baseline: 1.8395x; 1.2737x over previous
import jax
import jax.numpy as jnp
from jax import lax
from jax.experimental import pallas as pl
from jax.experimental.pallas import tpu as pltpu
from jax.experimental.pallas import tpu_sc as plsc

B = 16384
D_NUM = 13
N_CAT = 26
CARD = 1000
D_TOKEN = 64
TOTAL_CAT = N_CAT * CARD
N_TOK = 1 + D_NUM + N_CAT

NUM_ROWS = (1 + D_NUM) * D_TOKEN
OUT_ROWS = N_TOK * D_TOKEN

SC_CORES = 2
SC_SUBCORES = 16
NW = SC_CORES * SC_SUBCORES
COLS_PER_W = B // NW
BLK = 256
N_BLK = COLS_PER_W // BLK
N_UNIT = N_CAT * N_BLK
W_IDX = N_CAT * COLS_PER_W


TAB_W = 128


def _prep_kernel(emb_ref, bias_ref, xcat_ref, tab_ref, idx_ref):
    tab_ref[...] = emb_ref[...] + bias_ref[...]
    offs = lax.broadcasted_iota(jnp.int32, (N_CAT, B), 0) * CARD
    idx_ref[...] = xcat_ref[...] + offs


def _prep(cat_emb_p, bias_cat_p, x_cat_t):
    emb3 = cat_emb_p.reshape(N_CAT, CARD, TAB_W)
    bias3 = bias_cat_p.reshape(N_CAT, 1, TAB_W)
    tab3, idx = pl.pallas_call(
        _prep_kernel,
        out_shape=(
            jax.ShapeDtypeStruct((N_CAT, CARD, TAB_W), jnp.float32),
            jax.ShapeDtypeStruct((N_CAT, B), jnp.int32),
        ),
    )(emb3, bias3, x_cat_t)
    return tab3.reshape(TOTAL_CAT, TAB_W), idx.reshape(N_CAT * B)


def _sc_body(
    tab_ref,
    idx_ref,
    out_ref,
    idx_v,
    rows_a,
    rows_b,
    tp_a,
    tp_b,
    sg_a,
    sg_b,
    so_a,
    so_b,
):
    wid = lax.axis_index("s") * SC_CORES + lax.axis_index("c")
    col0 = wid * COLS_PER_W

    def stage(c, carry):
        pltpu.sync_copy(
            idx_ref.at[pl.ds(c * B + col0, COLS_PER_W)],
            idx_v.at[pl.ds(c * COLS_PER_W, COLS_PER_W)],
        )
        return carry

    lax.fori_loop(0, N_CAT, stage, 0)

    d_iotas = [lax.iota(jnp.int32, 16) + 16 * dk for dk in range(D_TOKEN // 16)]

    def start_gather(u, rows_v, sem):
        c = u // N_BLK
        j = u % N_BLK
        pltpu.async_copy(
            tab_ref.at[idx_v.at[pl.ds(c * COLS_PER_W + j * BLK, BLK)]],
            rows_v,
            sem,
        )

    def transpose(rows_v, tp_v):
        @plsc.parallel_loop(0, BLK, step=1, unroll=8)
        def _(r):
            rvec = jnp.full((16,), r, jnp.int32)
            for dk in range(D_TOKEN // 16):
                plsc.store_scatter(
                    tp_v,
                    [d_iotas[dk], rvec],
                    rows_v[r, pl.ds(16 * dk, 16)],
                )

    def out_slice(u):
        c = u // N_BLK
        j = u % N_BLK
        return out_ref.at[
            pl.ds(NUM_ROWS + c * D_TOKEN, D_TOKEN),
            pl.ds(col0 + j * BLK, BLK),
        ]

    def start_out(u, tp_v, sem):
        pltpu.async_copy(tp_v.at[:, pl.ds(0, BLK)], out_slice(u), sem)

    def wait_out(u, tp_v, sem):
        pltpu.make_async_copy(tp_v.at[:, pl.ds(0, BLK)], out_slice(u), sem).wait()

    def wait_gather(rows_v, sem):
        pltpu.make_async_copy(
            tab_ref.at[idx_v.at[pl.ds(0, BLK)]], rows_v, sem
        ).wait()

    start_gather(0, rows_a, sg_a)

    def pair(p, carry):
        u0 = 2 * p
        u1 = u0 + 1
        start_gather(u1, rows_b, sg_b)
        wait_gather(rows_a, sg_a)

        @pl.when(p > 0)
        def _():
            wait_out(u0, tp_a, so_a)

        transpose(rows_a, tp_a)
        start_out(u0, tp_a, so_a)

        @pl.when(p < N_UNIT // 2 - 1)
        def _():
            start_gather(u0 + 2, rows_a, sg_a)

        wait_gather(rows_b, sg_b)

        @pl.when(p > 0)
        def _():
            wait_out(u1, tp_b, so_b)

        transpose(rows_b, tp_b)
        start_out(u1, tp_b, so_b)
        return carry

    lax.fori_loop(0, N_UNIT // 2, pair, 0)
    wait_out(0, tp_a, so_a)
    wait_out(0, tp_b, so_b)


def _sc_gather(table, idx_flat):
    mesh = plsc.VectorSubcoreMesh(core_axis_name="c", subcore_axis_name="s")
    f = pl.kernel(
        _sc_body,
        out_type=jax.ShapeDtypeStruct((OUT_ROWS, B), jnp.float32),
        mesh=mesh,
        scratch_types=[
            pltpu.VMEM((W_IDX,), jnp.int32),
            pltpu.VMEM((BLK, TAB_W), jnp.float32),
            pltpu.VMEM((BLK, TAB_W), jnp.float32),
            pltpu.VMEM((D_TOKEN, BLK + 1), jnp.float32),
            pltpu.VMEM((D_TOKEN, BLK + 1), jnp.float32),
            pltpu.SemaphoreType.DMA,
            pltpu.SemaphoreType.DMA,
            pltpu.SemaphoreType.DMA,
            pltpu.SemaphoreType.DMA,
        ],
        compiler_params=pltpu.CompilerParams(needs_layout_passes=False),
    )
    return f(table, idx_flat)


def _num_kernel(x_ref, w_ref, b_ref, alias_ref, o_ref):
    del alias_ref
    o_ref[...] = (
        w_ref[0, 0][:, None] * x_ref[0, 0][None, :] + b_ref[0, 0][:, None]
    )


def _num_finish(xf_t, weight, bias_num, out0):
    bs = 1024
    w3 = weight.reshape(1 + D_NUM, 1, D_TOKEN)
    b3 = bias_num.reshape(1 + D_NUM, 1, D_TOKEN)
    xf3 = xf_t.reshape(1 + D_NUM, 1, B)
    return pl.pallas_call(
        _num_kernel,
        grid=(1 + D_NUM, B // bs),
        in_specs=[
            pl.BlockSpec((1, 1, bs), lambda t, i: (t, 0, i)),
            pl.BlockSpec((1, 1, D_TOKEN), lambda t, i: (t, 0, 0)),
            pl.BlockSpec((1, 1, D_TOKEN), lambda t, i: (t, 0, 0)),
            pl.BlockSpec(memory_space=pl.ANY),
        ],
        out_specs=pl.BlockSpec((D_TOKEN, bs), lambda t, i: (t, i)),
        out_shape=jax.ShapeDtypeStruct((OUT_ROWS, B), jnp.float32),
        input_output_aliases={3: 0},
    )(xf3, w3, b3, out0)


def kernel(x_num, x_cat, weight, bias, cat_emb):
    x_cat_t = x_cat.T
    xf_t = jnp.concatenate(
        [jnp.ones((1, B), jnp.float32), x_num.T], axis=0
    )
    bias_num = jnp.concatenate(
        [jnp.zeros((1, D_TOKEN), jnp.float32), bias[:D_NUM]], axis=0
    )
    cat_emb_p = jnp.pad(cat_emb, ((0, 0), (0, TAB_W - D_TOKEN)))
    bias_cat_p = jnp.pad(bias[D_NUM:], ((0, 0), (0, TAB_W - D_TOKEN)))
    table, idx_flat = _prep(cat_emb_p, bias_cat_p, x_cat_t)
    out0 = _sc_gather(table, idx_flat)
    out2d = _num_finish(xf_t, weight, bias_num, out0)
    return out2d.reshape(N_TOK, D_TOKEN, B).transpose(2, 0, 1)
